# baseline (speedup 1.0000x reference)
import functools

import jax
import jax.numpy as jnp
from jax import lax
from jax.experimental import pallas as pl
from jax.experimental.pallas import tpu as pltpu
from jax.experimental.pallas import tpu_sc as plsc


def kernel(x0, x1, indices):
    E, D = x1.shape
    info = plsc.get_sparse_core_info()
    NC, NS = info.num_cores, info.num_subcores
    NW = NC * NS
    assert E % NW == 0
    e_per_w = E // NW
    B = 80
    assert e_per_w % B == 0
    niter = e_per_w // B
    assert niter % 4 == 1

    mesh = plsc.VectorSubcoreMesh(core_axis_name="c", subcore_axis_name="s")

    @functools.partial(
        pl.kernel,
        mesh=mesh,
        out_type=jax.ShapeDtypeStruct((E, D), jnp.float32),
        scratch_types=(
            [pltpu.VMEM((e_per_w,), jnp.int32)]
            + [pltpu.VMEM((B, D), jnp.float32) for _ in range(4)]
            + [pltpu.VMEM((B, D), jnp.float32) for _ in range(4)]
            + [pltpu.VMEM((B, D), jnp.float32) for _ in range(2)]
            + [pltpu.SemaphoreType.DMA for _ in range(4)]
            + [pltpu.SemaphoreType.DMA for _ in range(4)]
            + [pltpu.SemaphoreType.DMA for _ in range(2)]
        ),
    )
    def run(x0_hbm, x1_hbm, idx_hbm, out_hbm,
            idx_v, w0, w1, w2, w3, y0, y1, y2, y3, o0, o1,
            g0, g1, g2, g3, p0, p1, p2, p3, q0, q1):
        wid = lax.axis_index("s") * NC + lax.axis_index("c")
        base = wid * e_per_w
        pltpu.sync_copy(idx_hbm.at[pl.ds(base, e_per_w)], idx_v)

        wbufs = (w0, w1, w2, w3)
        ybufs = (y0, y1, y2, y3)
        obufs = (o0, o1)
        gsems = (g0, g1, g2, g3)
        xsems = (p0, p1, p2, p3)
        osems = (q0, q1)

        H = B // 2

        def issue_inputs(j, s4):
            pltpu.async_copy(
                x0_hbm.at[idx_v.at[pl.ds(j * B, H)]],
                wbufs[s4].at[pl.ds(0, H)], gsems[s4])
            pltpu.async_copy(
                x0_hbm.at[idx_v.at[pl.ds(j * B + H, H)]],
                wbufs[s4].at[pl.ds(H, H)], gsems[s4])
            pltpu.async_copy(
                x1_hbm.at[pl.ds(base + j * B, H)],
                ybufs[s4].at[pl.ds(0, H)], xsems[s4])
            pltpu.async_copy(
                x1_hbm.at[pl.ds(base + j * B + H, H)],
                ybufs[s4].at[pl.ds(H, H)], xsems[s4])

        for jj in range(4):
            issue_inputs(jj, jj)

        def step(j, k):
            s4, s2 = k, k % 2
            w, y, o = wbufs[s4], ybufs[s4], obufs[s2]
            pltpu.make_async_copy(x1_hbm.at[pl.ds(0, B)], w, gsems[s4]).wait()
            pltpu.make_async_copy(x1_hbm.at[pl.ds(0, B)], y, xsems[s4]).wait()

            @pl.when(j >= 2)
            def _():
                pltpu.make_async_copy(o, out_hbm.at[pl.ds(0, B)], osems[s2]).wait()

            @plsc.parallel_loop(0, B, unroll=8)
            def row(r):
                for c in range(D // 16):
                    sl = pl.ds(c * 16, 16)
                    o[r, sl] = w[r, sl] * y[r, sl]

            pltpu.async_copy(o.at[pl.ds(0, H)],
                             out_hbm.at[pl.ds(base + j * B, H)], osems[s2])
            pltpu.async_copy(o.at[pl.ds(H, H)],
                             out_hbm.at[pl.ds(base + j * B + H, H)], osems[s2])

            @pl.when(j + 4 < niter)
            def _():
                issue_inputs(j + 4, s4)

        def outer(g, carry):
            for k in range(4):
                step(4 * g + k, k)
            return carry

        lax.fori_loop(0, (niter - 1) // 4, outer, 0)
        step(niter - 1, 0)

        pltpu.make_async_copy(o1, out_hbm.at[pl.ds(0, B)], osems[1]).wait()
        pltpu.make_async_copy(o0, out_hbm.at[pl.ds(0, B)], osems[0]).wait()

    return run(x0, x1, indices)


# speedup vs baseline: 1.0032x; 1.0032x over previous
import functools

import jax
import jax.numpy as jnp
from jax import lax
from jax.experimental import pallas as pl
from jax.experimental.pallas import tpu as pltpu
from jax.experimental.pallas import tpu_sc as plsc


def kernel(x0, x1, indices):
    E, D = x1.shape
    info = plsc.get_sparse_core_info()
    NC, NS = info.num_cores, info.num_subcores
    NW = NC * NS
    assert E % NW == 0
    e_per_w = E // NW
    B = 80
    assert e_per_w % B == 0
    niter = e_per_w // B
    assert niter % 4 == 1

    mesh = plsc.VectorSubcoreMesh(core_axis_name="c", subcore_axis_name="s")

    @functools.partial(
        pl.kernel,
        mesh=mesh,
        out_type=jax.ShapeDtypeStruct((E, D), jnp.float32),
        scratch_types=(
            [pltpu.VMEM((e_per_w,), jnp.int32)]
            + [pltpu.VMEM((B, D), jnp.float32) for _ in range(4)]
            + [pltpu.VMEM((B, D), jnp.float32) for _ in range(4)]
            + [pltpu.VMEM((B, D), jnp.float32) for _ in range(2)]
            + [pltpu.SemaphoreType.DMA for _ in range(4)]
            + [pltpu.SemaphoreType.DMA for _ in range(4)]
            + [pltpu.SemaphoreType.DMA for _ in range(2)]
        ),
    )
    def run(x0_hbm, x1_hbm, idx_hbm, out_hbm,
            idx_v, w0, w1, w2, w3, y0, y1, y2, y3, o0, o1,
            g0, g1, g2, g3, p0, p1, p2, p3, q0, q1):
        wid = lax.axis_index("s") * NC + lax.axis_index("c")
        base = wid * e_per_w
        pltpu.sync_copy(idx_hbm.at[pl.ds(base, e_per_w)], idx_v)

        wbufs = (w0, w1, w2, w3)
        ybufs = (y0, y1, y2, y3)
        obufs = (o0, o1)
        gsems = (g0, g1, g2, g3)
        xsems = (p0, p1, p2, p3)
        osems = (q0, q1)

        def issue_inputs(j, s4):
            pltpu.async_copy(
                x0_hbm.at[idx_v.at[pl.ds(j * B, B)]], wbufs[s4], gsems[s4])
            pltpu.async_copy(
                x1_hbm.at[pl.ds(base + j * B, B)], ybufs[s4], xsems[s4])

        for jj in range(4):
            issue_inputs(jj, jj)

        def step(j, k):
            s4, s2 = k, k % 2
            w, y, o = wbufs[s4], ybufs[s4], obufs[s2]
            pltpu.make_async_copy(x1_hbm.at[pl.ds(0, B)], w, gsems[s4]).wait()
            pltpu.make_async_copy(x1_hbm.at[pl.ds(0, B)], y, xsems[s4]).wait()

            @pl.when(j >= 2)
            def _():
                pltpu.make_async_copy(o, out_hbm.at[pl.ds(0, B)], osems[s2]).wait()

            @plsc.parallel_loop(0, B, unroll=8)
            def row(r):
                for c in range(D // 16):
                    sl = pl.ds(c * 16, 16)
                    o[r, sl] = w[r, sl] * y[r, sl]

            pltpu.async_copy(o, out_hbm.at[pl.ds(base + j * B, B)], osems[s2])

            @pl.when(j + 4 < niter)
            def _():
                issue_inputs(j + 4, s4)

        def outer(g, carry):
            for k in range(4):
                step(4 * g + k, k)
            return carry

        lax.fori_loop(0, (niter - 1) // 4, outer, 0)
        step(niter - 1, 0)

        pltpu.make_async_copy(o1, out_hbm.at[pl.ds(0, B)], osems[1]).wait()
        pltpu.make_async_copy(o0, out_hbm.at[pl.ds(0, B)], osems[0]).wait()

    return run(x0, x1, indices)
